# baseline (speedup 1.0000x reference)
import functools

import jax
import jax.numpy as jnp
from jax import lax
from jax.experimental import pallas as pl
from jax.experimental.pallas import tpu as pltpu
from jax.experimental.pallas import tpu_sc as plsc

N = 128
NC, NS = 2, 16
NW = NC * NS
TOT = 4096 * 200
CPW = TOT // NW
C = 640
NCHUNK = CPW // C
NSUB = C // 128
NA = 4

_mesh = plsc.VectorSubcoreMesh(core_axis_name="c", subcore_axis_name="s")


@functools.partial(
    pl.kernel,
    mesh=_mesh,
    out_type=jax.ShapeDtypeStruct((TOT * N,), jnp.float32),
    scratch_types=[
        pltpu.VMEM((CPW,), jnp.int32),
        pltpu.VMEM((C * N,), jnp.float32),
        pltpu.VMEM((2 * NSUB, 128), jnp.int32),
        pltpu.VMEM((NSUB, 128), jnp.float32),
        pltpu.SemaphoreType.DMA,
        pltpu.SemaphoreType.DMA,
        pltpu.SemaphoreType.DMA,
        pltpu.SemaphoreType.DMA,
        pltpu.SemaphoreType.DMA,
        pltpu.SemaphoreType.DMA,
    ],
    compiler_params=pltpu.CompilerParams(needs_layout_passes=False),
)
def _one_hot_sc(idx_hbm, zeros_hbm, ones_hbm, out_hbm,
                idx_v, zbuf, offs, ones_v, a0, a1, a2, a3, b0, b1):
    wid = lax.axis_index("s") * NC + lax.axis_index("c")
    lane = lax.iota(jnp.int32, 16)
    wbase = wid * CPW
    asems = (a0, a1, a2, a3)
    bsems = (b0, b1)

    pltpu.sync_copy(idx_hbm.at[pl.ds(wbase, CPW)], idx_v)
    pltpu.sync_copy(zeros_hbm, zbuf)
    pltpu.sync_copy(ones_hbm, ones_v)

    def zstream(c, sem):
        pltpu.async_copy(
            zbuf, out_hbm.at[pl.ds((wbase + c * C) * N, C * N)], sem)

    def await_z(sem):
        pltpu.make_async_copy(
            zbuf, out_hbm.at[pl.ds(wbase * N, C * N)], sem).wait()

    def build_offs(q, c):
        base = (wbase + c * C) * N
        for g in range(C // 16):
            off16 = idx_v[pl.ds(c * C + g * 16, 16)] + (
                base + (g * 16) * N) + lane * N
            offs[q * NSUB + g // 8, pl.ds((g % 8) * 16, 16)] = off16

    def scatter_ones(q, sem):
        for s in range(NSUB):
            pltpu.async_copy(
                ones_v.at[s], out_hbm.at[offs.at[q * NSUB + s]], sem)

    def await_ones(q, sem):
        for s in range(NSUB):
            pltpu.make_async_copy(
                ones_v.at[s], out_hbm.at[offs.at[q * NSUB + s]], sem).wait()

    def quad(p, carry):
        for q in range(NA):
            c = p * NA + q

            @pl.when(c >= NA)
            def _reuse_sem():
                await_z(asems[q])
            zstream(c, asems[q])
        return carry

    lax.fori_loop(0, NCHUNK // NA, quad, 0)
    for q in range(NA):
        await_z(asems[q])

    def ones_pair(p, carry):
        for h in range(2):
            c = p * 2 + h

            @pl.when(c >= 2)
            def _reuse():
                await_ones(h, bsems[h])
            build_offs(h, c)
            scatter_ones(h, bsems[h])
        return carry

    lax.fori_loop(0, NCHUNK // 2, ones_pair, 0)
    for h in range(2):
        await_ones(h, bsems[h])


def kernel(Z, eye):
    del eye
    idx = Z.reshape(-1).astype(jnp.int32)
    zeros = jnp.zeros((C * N,), jnp.float32)
    ones = jnp.ones((NSUB, 128), jnp.float32)
    out = _one_hot_sc(idx, zeros, ones)
    return out.reshape(Z.shape + (N,))


# speedup vs baseline: 5.4244x; 5.4244x over previous
import functools

import jax
import jax.numpy as jnp
from jax import lax
from jax.experimental import pallas as pl
from jax.experimental.pallas import tpu as pltpu
from jax.experimental.pallas import tpu_sc as plsc

N = 128
NC, NS = 2, 16
NW = NC * NS
TOT = 4096 * 200
CPW = TOT // NW
C = 128
NBUF = 5
NCHUNK = CPW // C
NGRP = NCHUNK // NBUF

_mesh = plsc.VectorSubcoreMesh(core_axis_name="c", subcore_axis_name="s")


@functools.partial(
    pl.kernel,
    mesh=_mesh,
    out_type=jax.ShapeDtypeStruct((TOT, N), jnp.float32),
    scratch_types=[
        pltpu.VMEM((CPW,), jnp.int32),
    ] + [pltpu.VMEM((C, N), jnp.float32)] * NBUF
      + [pltpu.SemaphoreType.DMA] * (NBUF + 1),
    compiler_params=pltpu.CompilerParams(needs_layout_passes=False),
)
def _one_hot_sc(idx_hbm, zeros_hbm, out_hbm, idx_v, *bufsem):
    rows_bufs = bufsem[:NBUF]
    sems = bufsem[NBUF:2 * NBUF]
    sem_i = bufsem[2 * NBUF]
    wid = lax.axis_index("s") * NC + lax.axis_index("c")
    lane = lax.iota(jnp.int32, 16)
    ones = jnp.full((16,), 1.0, jnp.float32)
    zeros = jnp.zeros((16,), jnp.float32)
    wbase = wid * CPW

    pltpu.async_copy(idx_hbm.at[pl.ds(wbase, CPW)], idx_v, sem_i)
    for buf, sem in zip(rows_bufs, sems):
        pltpu.async_copy(zeros_hbm, buf, sem)
    pltpu.make_async_copy(idx_hbm.at[pl.ds(wbase, CPW)], idx_v, sem_i).wait()
    for buf, sem in zip(rows_bufs, sems):
        pltpu.make_async_copy(zeros_hbm, buf, sem).wait()

    def scatter(buf, c, val):
        for i in range(C // 16):
            rows = lane + i * 16
            cols = idx_v[pl.ds(c * C + i * 16, 16)]
            plsc.store_scatter(buf, [rows, cols], val)

    def group(p, carry):
        for q, (buf, sem) in enumerate(zip(rows_bufs, sems)):
            c = p * NBUF + q

            @pl.when(p > 0)
            def _recycle():
                pltpu.make_async_copy(
                    buf, out_hbm.at[pl.ds(wbase, C)], sem).wait()
                scatter(buf, c - NBUF, zeros)

            scatter(buf, c, ones)
            pltpu.async_copy(buf, out_hbm.at[pl.ds(wbase + c * C, C)], sem)
        return carry

    lax.fori_loop(0, NGRP, group, 0)
    for buf, sem in zip(rows_bufs, sems):
        pltpu.make_async_copy(buf, out_hbm.at[pl.ds(wbase, C)], sem).wait()


def kernel(Z, eye):
    del eye
    idx = Z.reshape(-1).astype(jnp.int32)
    zeros = jnp.zeros((C, N), jnp.float32)
    out = _one_hot_sc(idx, zeros)
    return out.reshape(Z.shape + (N,))


# speedup vs baseline: 5.4269x; 1.0005x over previous
import functools

import jax
import jax.numpy as jnp
from jax import lax
from jax.experimental import pallas as pl
from jax.experimental.pallas import tpu as pltpu
from jax.experimental.pallas import tpu_sc as plsc

N = 128
NC, NS = 2, 16
NW = NC * NS
TOT = 4096 * 200
CPW = TOT // NW
C = 128
NBUF = 5
NCHUNK = CPW // C
NGRP = NCHUNK // NBUF

_mesh = plsc.VectorSubcoreMesh(core_axis_name="c", subcore_axis_name="s")


@functools.partial(
    pl.kernel,
    mesh=_mesh,
    out_type=jax.ShapeDtypeStruct((TOT, N), jnp.float32),
    scratch_types=[
        pltpu.VMEM((CPW,), jnp.int32),
    ] + [pltpu.VMEM((C, N), jnp.float32)] * NBUF
      + [pltpu.SemaphoreType.DMA] * (NBUF + 1),
    compiler_params=pltpu.CompilerParams(
        needs_layout_passes=False, use_tc_tiling_on_sc=False),
)
def _one_hot_sc(idx_hbm, zeros_hbm, out_hbm, idx_v, *bufsem):
    rows_bufs = bufsem[:NBUF]
    sems = bufsem[NBUF:2 * NBUF]
    sem_i = bufsem[2 * NBUF]
    wid = lax.axis_index("s") * NC + lax.axis_index("c")
    lane = lax.iota(jnp.int32, 16)
    ones = jnp.full((16,), 1.0, jnp.float32)
    zeros = jnp.zeros((16,), jnp.float32)
    wbase = wid * CPW

    pltpu.async_copy(idx_hbm.at[pl.ds(wbase, CPW)], idx_v, sem_i)
    for buf, sem in zip(rows_bufs, sems):
        pltpu.async_copy(zeros_hbm, buf, sem)
    pltpu.make_async_copy(idx_hbm.at[pl.ds(wbase, CPW)], idx_v, sem_i).wait()
    for buf, sem in zip(rows_bufs, sems):
        pltpu.make_async_copy(zeros_hbm, buf, sem).wait()

    def scatter(buf, c, val):
        for i in range(C // 16):
            rows = lane + i * 16
            cols = idx_v[pl.ds(c * C + i * 16, 16)]
            plsc.store_scatter(buf, [rows, cols], val)

    def group(p, carry):
        for q, (buf, sem) in enumerate(zip(rows_bufs, sems)):
            c = p * NBUF + q

            @pl.when(p > 0)
            def _recycle():
                pltpu.make_async_copy(
                    buf, out_hbm.at[pl.ds(wbase, C)], sem).wait()
                scatter(buf, c - NBUF, zeros)

            scatter(buf, c, ones)
            pltpu.async_copy(buf, out_hbm.at[pl.ds(wbase + c * C, C)], sem)
        return carry

    lax.fori_loop(0, NGRP, group, 0)
    for buf, sem in zip(rows_bufs, sems):
        pltpu.make_async_copy(buf, out_hbm.at[pl.ds(wbase, C)], sem).wait()


def kernel(Z, eye):
    del eye
    idx = Z.reshape(-1).astype(jnp.int32)
    zeros = jnp.zeros((C, N), jnp.float32)
    out = _one_hot_sc(idx, zeros)
    return out.reshape(Z.shape + (N,))
